# Initial kernel scaffold; baseline (speedup 1.0000x reference)
import jax
import jax.numpy as jnp
from jax.experimental import pallas as pl
from jax.experimental.pallas import tpu as pltpu
from jax.experimental.pallas import tpu_sc as plsc

N = 16384
K = 8192
C = 64
NB = 512
KB = 2048
GW = 128


def _argmin_body(x_ref, cb_ref, idx_ref):
    x = x_ref[...]
    x2 = jnp.sum(x * x, axis=1, keepdims=True)

    def step(k, carry):
        best, bidx = carry
        cb = cb_ref[pl.ds(k * KB, KB), :]
        c2 = jnp.sum(cb * cb, axis=1)[None, :]
        xc = jax.lax.dot_general(
            x, cb, (((1,), (1,)), ((), ())),
            preferred_element_type=jnp.float32,
            precision=jax.lax.Precision.HIGHEST)
        d = x2 - 2.0 * xc
        d = d + c2
        m = jnp.min(d, axis=1, keepdims=True)
        ii = jax.lax.broadcasted_iota(jnp.int32, d.shape, 1) + k * KB
        cand = jnp.min(jnp.where(d == m, ii, K), axis=1, keepdims=True)
        upd = m < best
        return jnp.where(upd, m, best), jnp.where(upd, cand, bidx)

    best0 = jnp.full((NB, 1), jnp.inf, jnp.float32)
    bidx0 = jnp.zeros((NB, 1), jnp.int32)
    _, bidx = jax.lax.fori_loop(0, K // KB, step, (best0, bidx0))
    idx_ref[...] = bidx[:, 0]


def _nearest_code(x, codebook):
    return pl.pallas_call(
        _argmin_body,
        grid=(N // NB,),
        in_specs=[
            pl.BlockSpec((NB, C), lambda i: (i, 0)),
            pl.BlockSpec((K, C), lambda i: (0, 0)),
        ],
        out_specs=pl.BlockSpec((NB,), lambda i: (i,)),
        out_shape=jax.ShapeDtypeStruct((N,), jnp.int32),
    )(x, codebook)


def _sc_gather(codebook, idx):
    idx2 = idx.reshape((1, N))
    mesh = plsc.VectorSubcoreMesh(core_axis_name="c", subcore_axis_name="s")

    @pl.kernel(out_type=jax.ShapeDtypeStruct((N, C), codebook.dtype),
               mesh=mesh)
    def gkernel(cb_hbm, i_hbm, o_hbm):
        def body(i_vmem, o_vmem):
            pltpu.sync_copy(cb_hbm.at[i_vmem.at[0]], o_vmem)

        pltpu.emit_pipeline(
            body,
            grid=(N // GW,),
            in_specs=[pl.BlockSpec((1, GW), lambda i: (0, i))],
            out_specs=[pl.BlockSpec((GW, C), lambda i: (i, 0))],
            core_axis_name=("c", "s"),
            dimension_semantics=(pltpu.PARALLEL,),
        )(i_hbm, o_hbm)

    return gkernel(codebook, idx2)


def kernel(x, codebook):
    idx = _nearest_code(x, codebook)
    quantized = _sc_gather(codebook, idx)
    return quantized, idx


# speedup vs baseline: 1.1060x; 1.1060x over previous
import jax
import jax.numpy as jnp
from jax.experimental import pallas as pl
from jax.experimental.pallas import tpu as pltpu
from jax.experimental.pallas import tpu_sc as plsc

N = 16384
K = 8192
C = 64
NB = 256
GW = 128


def _rowsum64(sq):
    g = sq[:, 0:8]
    for v in range(1, 8):
        g = g + sq[:, 8 * v:8 * v + 8]
    h = g[:, 0:4] + g[:, 4:8]
    p = h[:, 0:2] + h[:, 2:4]
    return p[:, 0:1] + p[:, 1:2]


def _c2_body(cb_ref, c2_ref):
    cc = cb_ref[...]
    c2_ref[...] = _rowsum64(cc * cc)


def _codebook_norms(codebook):
    return pl.pallas_call(
        _c2_body,
        out_shape=jax.ShapeDtypeStruct((K, 1), jnp.float32),
    )(codebook)


def _argmin_body(x_ref, cb_ref, c2_ref, idx_ref):
    x = x_ref[...]
    x2 = _rowsum64(x * x)
    cb = cb_ref[...]
    c2 = c2_ref[...]
    xc = jax.lax.dot_general(
        x.astype(jnp.bfloat16), cb.astype(jnp.bfloat16),
        (((1,), (1,)), ((), ())),
        preferred_element_type=jnp.float32)
    d = (x2 - 2.0 * xc) + c2
    m = jnp.min(d, axis=1, keepdims=True)
    ii = jax.lax.broadcasted_iota(jnp.int32, d.shape, 1)
    cand = jnp.min(jnp.where(d == m, ii, K), axis=1, keepdims=True)
    idx_ref[...] = cand[:, 0]


def _nearest_code(x, codebook, c2_row):
    return pl.pallas_call(
        _argmin_body,
        grid=(N // NB,),
        in_specs=[
            pl.BlockSpec((NB, C), lambda i: (i, 0)),
            pl.BlockSpec((K, C), lambda i: (0, 0)),
            pl.BlockSpec((1, K), lambda i: (0, 0)),
        ],
        out_specs=pl.BlockSpec((NB,), lambda i: (i,)),
        out_shape=jax.ShapeDtypeStruct((N,), jnp.int32),
    )(x, codebook, c2_row)


def _sc_gather(codebook_pad, idx):
    idx2 = idx.reshape((1, N))
    mesh = plsc.VectorSubcoreMesh(core_axis_name="c", subcore_axis_name="s")
    cp = codebook_pad.shape[1]

    @pl.kernel(out_type=jax.ShapeDtypeStruct((N, cp), codebook_pad.dtype),
               mesh=mesh)
    def gkernel(cb_hbm, i_hbm, o_hbm):
        def body(i_vmem, o_vmem):
            pltpu.sync_copy(cb_hbm.at[i_vmem.at[0]], o_vmem)

        pltpu.emit_pipeline(
            body,
            grid=(N // GW,),
            in_specs=[pl.BlockSpec((1, GW), lambda i: (0, i))],
            out_specs=[pl.BlockSpec((GW, cp), lambda i: (i, 0))],
            core_axis_name=("c", "s"),
            dimension_semantics=(pltpu.PARALLEL,),
        )(i_hbm, o_hbm)

    return gkernel(codebook_pad, idx2)


def kernel(x, codebook):
    c2_row = _codebook_norms(codebook).reshape((1, K))
    idx = _nearest_code(x, codebook, c2_row)
    codebook_pad = jnp.concatenate([codebook, jnp.zeros_like(codebook)], axis=1)
    quantized = _sc_gather(codebook_pad, idx)[:, :C]
    return quantized, idx
